# baseline (speedup 1.0000x reference)
import jax
import jax.numpy as jnp
from jax.experimental import pallas as pl


def _bcast_body(tab_ref, out_ref):
    out_ref[...] = jnp.broadcast_to(tab_ref[...][None, :, :], out_ref.shape)


def kernel(inputs, position_embeddings):
    batch, seq_len, d_model = inputs.shape
    block_s = 256
    grid = (seq_len // block_s,)
    out = pl.pallas_call(
        _bcast_body,
        grid=grid,
        in_specs=[
            pl.BlockSpec((block_s, d_model), lambda i: (i, 0)),
        ],
        out_specs=pl.BlockSpec((batch, block_s, d_model), lambda i: (0, i, 0)),
        out_shape=jax.ShapeDtypeStruct((batch, seq_len, d_model), position_embeddings.dtype),
    )(position_embeddings)
    return out


# speedup vs baseline: 1.1263x; 1.1263x over previous
import jax
import jax.numpy as jnp
from jax.experimental import pallas as pl


def _bcast_body(tab_ref, out_ref):
    out_ref[...] = jnp.broadcast_to(tab_ref[...][None, :, :], out_ref.shape)


def kernel(inputs, position_embeddings):
    batch, seq_len, d_model = inputs.shape
    block_s = 1024
    grid = (seq_len // block_s,)
    out = pl.pallas_call(
        _bcast_body,
        grid=grid,
        in_specs=[
            pl.BlockSpec((block_s, d_model), lambda i: (i, 0)),
        ],
        out_specs=pl.BlockSpec((batch, block_s, d_model), lambda i: (0, i, 0)),
        out_shape=jax.ShapeDtypeStruct((batch, seq_len, d_model), position_embeddings.dtype),
    )(position_embeddings)
    return out
